# baseline (speedup 1.0000x reference)
import functools

import jax
import jax.numpy as jnp
from jax import lax
from jax.experimental import pallas as pl
from jax.experimental.pallas import tpu as pltpu
from jax.experimental.pallas import tpu_sc as plsc

_NBUK = 65536
_LANES = 16


def _encode_body(x_ref, w_ref, be_ref, bd_ref, acts_ref):
    xm = x_ref[...] - bd_ref[...]
    a = lax.dot_general(xm, w_ref[...], (((1,), (1,)), ((), ())),
                        preferred_element_type=jnp.float32)
    acts_ref[...] = jnp.maximum(a + be_ref[...], 0.0)


def _encode(x, w_enc, b_enc, b_dec, ft, f_lo, f_hi):
    b, d = x.shape
    f = w_enc.shape[0]
    off = f_lo // ft
    grid = ((f_hi - f_lo) // ft,)
    return pl.pallas_call(
        _encode_body,
        grid=grid,
        in_specs=[
            pl.BlockSpec((b, d), lambda i: (0, 0)),
            pl.BlockSpec((ft, d), lambda i: (i + off, 0)),
            pl.BlockSpec((1, ft), lambda i: (0, i + off)),
            pl.BlockSpec((1, d), lambda i: (0, 0)),
        ],
        out_specs=pl.BlockSpec((b, ft), lambda i: (0, i)),
        out_shape=jax.ShapeDtypeStruct((b, f_hi - f_lo), jnp.float32),
    )(x, w_enc, b_enc.reshape(1, f), b_dec.reshape(1, d))


_NBUK_A = 32768
_ROWS_PER_W = 8
_CCOLS = 2048


def _sc_mesh():
    return plsc.VectorSubcoreMesh(core_axis_name="c", subcore_axis_name="s")


def _zero_ref(ref, nwords):
    zeros = jnp.zeros((_LANES,), jnp.int32)

    @plsc.parallel_loop(0, nwords // _LANES, unroll=8)
    def _(i):
        ref[pl.ds(i * _LANES, _LANES)] = zeros


def _scan_chunks(acts_hbm, row0, ncols, buf0, buf1, sem0, sem1, process):
    n_chunks = ncols // _CCOLS

    def copy(c, buf, sem):
        return pltpu.make_async_copy(
            acts_hbm.at[pl.ds(row0, _ROWS_PER_W), pl.ds(c * _CCOLS, _CCOLS)],
            buf, sem)

    copy(0, buf0, sem0).start()

    def cbody(i, carry):
        for par, (buf_a, sem_a, buf_b, sem_b) in (
                (0, (buf0, sem0, buf1, sem1)),
                (1, (buf1, sem1, buf0, sem0))):
            c = i * 2 + par

            @pl.when(c + 1 < n_chunks)
            def _():
                copy(c + 1, buf_b, sem_b).start()

            copy(c, buf_a, sem_a).wait()
            process(buf_a)
        return carry

    lax.fori_loop(0, n_chunks // 2, cbody, 0)


def _make_hist_a(b, f):
    info = plsc.get_sparse_core_info()
    nw = info.num_cores * info.num_subcores

    @functools.partial(
        pl.kernel,
        mesh=_sc_mesh(),
        out_type=jax.ShapeDtypeStruct((nw, _NBUK_A), jnp.int32),
        compiler_params=pltpu.CompilerParams(needs_layout_passes=False),
        scratch_types=[
            pltpu.VMEM((_ROWS_PER_W, _CCOLS), jnp.float32),
            pltpu.VMEM((_ROWS_PER_W, _CCOLS), jnp.float32),
            pltpu.VMEM((_NBUK_A,), jnp.int32),
            pltpu.VMEM((_NBUK_A,), jnp.int32),
            pltpu.SemaphoreType.DMA,
            pltpu.SemaphoreType.DMA,
        ],
    )
    def hist_a(acts_hbm, out_hbm, buf0, buf1, h0, h1, sem0, sem1):
        wid = lax.axis_index("s") * info.num_cores + lax.axis_index("c")
        ones = jnp.ones((_LANES,), jnp.int32)
        zerosv = jnp.zeros((_LANES,), jnp.int32)
        _zero_ref(h0, _NBUK_A)
        _zero_ref(h1, _NBUK_A)

        def process(buf):
            @plsc.parallel_loop(0, _CCOLS // _LANES, unroll=4)
            def _(j):
                for r in range(_ROWS_PER_W):
                    bits = plsc.bitcast(buf[r, pl.ds(j * _LANES, _LANES)],
                                        jnp.int32)
                    buk = lax.shift_right_logical(bits, 16)
                    hist = h0 if r % 2 == 0 else h1
                    plsc.addupdate_scatter(hist, [buk], ones,
                                           mask=bits != zerosv)

        _scan_chunks(acts_hbm, wid * _ROWS_PER_W, f, buf0, buf1, sem0, sem1,
                     process)

        @plsc.parallel_loop(0, _NBUK_A // _LANES, unroll=8)
        def _(i):
            sl = pl.ds(i * _LANES, _LANES)
            h0[sl] = h0[sl] + h1[sl]

        pltpu.sync_copy(h0, out_hbm.at[wid])

    return hist_a


def _make_hist_b(b, f):
    info = plsc.get_sparse_core_info()
    nw = info.num_cores * info.num_subcores

    @functools.partial(
        pl.kernel,
        mesh=_sc_mesh(),
        out_type=jax.ShapeDtypeStruct((nw, _NBUK), jnp.int32),
        compiler_params=pltpu.CompilerParams(needs_layout_passes=False),
        scratch_types=[
            pltpu.VMEM((_ROWS_PER_W, _CCOLS), jnp.float32),
            pltpu.VMEM((_ROWS_PER_W, _CCOLS), jnp.float32),
            pltpu.VMEM((_NBUK,), jnp.int32),
            pltpu.VMEM((_LANES,), jnp.int32),
            pltpu.SemaphoreType.DMA,
            pltpu.SemaphoreType.DMA,
        ],
    )
    def hist_b(acts_hbm, hstar_hbm, out_hbm, buf0, buf1, h0, hsv, sem0, sem1):
        wid = lax.axis_index("s") * info.num_cores + lax.axis_index("c")
        pltpu.sync_copy(hstar_hbm, hsv)
        h16 = hsv[...]
        ones = jnp.ones((_LANES,), jnp.int32)
        lowmask = jnp.full((_LANES,), 0xFFFF, jnp.int32)
        _zero_ref(h0, _NBUK)

        def process(buf):
            @plsc.parallel_loop(0, _CCOLS // _LANES, unroll=4)
            def _(j):
                for r in range(_ROWS_PER_W):
                    bits = plsc.bitcast(buf[r, pl.ds(j * _LANES, _LANES)],
                                        jnp.int32)
                    match = lax.shift_right_logical(bits, 16) == h16
                    buk = bits & lowmask
                    plsc.addupdate_scatter(h0, [buk], ones, mask=match)

        _scan_chunks(acts_hbm, wid * _ROWS_PER_W, f, buf0, buf1, sem0, sem1,
                     process)
        pltpu.sync_copy(h0, out_hbm.at[wid])

    return hist_b


def _find_rank(cnt, kb):
    n = cnt.shape[0]
    nb = 256
    blk = n // nb
    c2 = cnt.reshape(nb, blk)
    rowsum = c2.sum(axis=1)
    srow = jnp.cumsum(rowsum[::-1])[::-1]
    srow_pad = jnp.concatenate([srow, jnp.zeros((1,), srow.dtype)])
    i = jnp.sum(srow >= kb).astype(jnp.int32) - 1
    i0 = jnp.maximum(i, 0)
    crow = c2[i0]
    w = jnp.cumsum(crow[::-1])[::-1]
    w_pad = jnp.concatenate([w, jnp.zeros((1,), w.dtype)])
    tail = srow_pad[i0 + 1]
    j = jnp.sum(tail + w >= kb).astype(jnp.int32) - 1
    j0 = jnp.maximum(j, 0)
    idx = i0 * blk + j0
    r = kb - (tail + w_pad[j0 + 1])
    return jnp.where(i < 0, jnp.int32(-1), idx), r


def _rank_threshold(acts_parts, kb):
    b = acts_parts[0].shape[0]
    cnt_a = sum(_make_hist_a(b, a.shape[1])(a).sum(axis=0)
                for a in acts_parts)
    hstar, r = _find_rank(cnt_a, kb)

    h16 = jnp.full((_LANES,), jnp.maximum(hstar, 0), jnp.int32)
    cnt_b = sum(_make_hist_b(b, a.shape[1])(a, h16).sum(axis=0)
                for a in acts_parts)
    lstar, _ = _find_rank(cnt_b, r)

    vk_bits = jnp.left_shift(jnp.maximum(hstar, 0), 16) | jnp.maximum(lstar, 0)
    vk = lax.bitcast_convert_type(vk_bits, jnp.float32)
    return jnp.where(hstar < 0, jnp.float32(0.0), vk)


def _decode_body(vk_ref, acts_ref, w_ref, init_ref, out_ref):
    i = pl.program_id(0)
    vk = vk_ref[0, 0]
    a = acts_ref[...]
    m = jnp.where(a >= vk, a, 0.0)
    part = lax.dot_general(m, w_ref[...], (((1,), (1,)), ((), ())),
                           preferred_element_type=jnp.float32)

    @pl.when(i == 0)
    def _():
        out_ref[...] = jnp.broadcast_to(init_ref[...], out_ref.shape)

    out_ref[...] += part


def _decode(vk, acts, w_dec, init, ft, f_lo):
    b, fs = acts.shape
    d = w_dec.shape[0]
    off = f_lo // ft
    grid = (fs // ft,)
    return pl.pallas_call(
        _decode_body,
        grid=grid,
        in_specs=[
            pl.BlockSpec(memory_space=pltpu.SMEM),
            pl.BlockSpec((b, ft), lambda i: (0, i)),
            pl.BlockSpec((d, ft), lambda i: (0, i + off)),
            pl.BlockSpec(init.shape, lambda i: (0, 0)),
        ],
        out_specs=pl.BlockSpec((b, d), lambda i: (0, 0)),
        out_shape=jax.ShapeDtypeStruct((b, d), jnp.float32),
    )(vk.reshape(1, 1), acts, w_dec, init)


def kernel(x, W_enc, b_enc, W_dec, b_dec):
    b, d = x.shape
    f = W_enc.shape[0]
    kb = min(64 * b, b * f)
    ft = 2048
    fh = f // 2
    acts0 = _encode(x, W_enc, b_enc, b_dec, ft, 0, fh)
    acts1 = _encode(x, W_enc, b_enc, b_dec, ft, fh, f)
    vk = _rank_threshold([acts0, acts1], kb)
    part = _decode(vk, acts0, W_dec, b_dec.reshape(1, d), ft, 0)
    return _decode(vk, acts1, W_dec, part, ft, fh)


# speedup vs baseline: 1.0841x; 1.0841x over previous
import functools

import jax
import jax.numpy as jnp
from jax import lax
from jax.experimental import pallas as pl
from jax.experimental.pallas import tpu as pltpu
from jax.experimental.pallas import tpu_sc as plsc

_NBUK = 65536
_LANES = 16


def _encode_body(x_ref, w_ref, be_ref, bd_ref, acts_ref):
    xm = x_ref[...] - bd_ref[...]
    a = lax.dot_general(xm, w_ref[...], (((1,), (1,)), ((), ())),
                        preferred_element_type=jnp.float32)
    acts_ref[...] = jnp.maximum(a + be_ref[...], 0.0)


def _encode(x, w_enc, b_enc, b_dec, ft, f_lo, f_hi):
    b, d = x.shape
    f = w_enc.shape[0]
    off = f_lo // ft
    grid = ((f_hi - f_lo) // ft,)
    return pl.pallas_call(
        _encode_body,
        grid=grid,
        in_specs=[
            pl.BlockSpec((b, d), lambda i: (0, 0)),
            pl.BlockSpec((ft, d), lambda i: (i + off, 0)),
            pl.BlockSpec((1, ft), lambda i: (0, i + off)),
            pl.BlockSpec((1, d), lambda i: (0, 0)),
        ],
        out_specs=pl.BlockSpec((b, ft), lambda i: (0, i)),
        out_shape=jax.ShapeDtypeStruct((b, f_hi - f_lo), jnp.float32),
    )(x, w_enc, b_enc.reshape(1, f), b_dec.reshape(1, d))


_NBUK_A = 32768
_ROWS_PER_W = 8
_CCOLS = 2048


def _sc_mesh():
    return plsc.VectorSubcoreMesh(core_axis_name="c", subcore_axis_name="s")


def _zero_ref(ref, nwords):
    zeros = jnp.zeros((_LANES,), jnp.int32)

    @plsc.parallel_loop(0, nwords // _LANES, unroll=8)
    def _(i):
        ref[pl.ds(i * _LANES, _LANES)] = zeros


def _scan_chunks(acts_hbm, row0, ncols, buf0, buf1, sem0, sem1, process):
    n_chunks = ncols // _CCOLS

    def copy(c, buf, sem):
        return pltpu.make_async_copy(
            acts_hbm.at[pl.ds(row0, _ROWS_PER_W), pl.ds(c * _CCOLS, _CCOLS)],
            buf, sem)

    copy(0, buf0, sem0).start()

    def cbody(i, carry):
        for par, (buf_a, sem_a, buf_b, sem_b) in (
                (0, (buf0, sem0, buf1, sem1)),
                (1, (buf1, sem1, buf0, sem0))):
            c = i * 2 + par

            @pl.when(c + 1 < n_chunks)
            def _():
                copy(c + 1, buf_b, sem_b).start()

            copy(c, buf_a, sem_a).wait()
            process(buf_a)
        return carry

    lax.fori_loop(0, n_chunks // 2, cbody, 0)


def _make_hist_a(b, f):
    info = plsc.get_sparse_core_info()
    nw = info.num_cores * info.num_subcores

    @functools.partial(
        pl.kernel,
        mesh=_sc_mesh(),
        out_type=jax.ShapeDtypeStruct((nw, _NBUK_A), jnp.int32),
        compiler_params=pltpu.CompilerParams(needs_layout_passes=False),
        scratch_types=[
            pltpu.VMEM((_ROWS_PER_W, _CCOLS), jnp.float32),
            pltpu.VMEM((_ROWS_PER_W, _CCOLS), jnp.float32),
            pltpu.VMEM((_NBUK_A,), jnp.int32),
            pltpu.VMEM((_NBUK_A,), jnp.int32),
            pltpu.SemaphoreType.DMA,
            pltpu.SemaphoreType.DMA,
        ],
    )
    def hist_a(acts_hbm, out_hbm, buf0, buf1, h0, h1, sem0, sem1):
        wid = lax.axis_index("s") * info.num_cores + lax.axis_index("c")
        ones = jnp.ones((_LANES,), jnp.int32)
        zerosv = jnp.zeros((_LANES,), jnp.int32)
        _zero_ref(h0, _NBUK_A)
        _zero_ref(h1, _NBUK_A)

        def process(buf):
            @plsc.parallel_loop(0, _CCOLS // _LANES, unroll=8)
            def _(j):
                for r in range(_ROWS_PER_W):
                    bits = plsc.bitcast(buf[r, pl.ds(j * _LANES, _LANES)],
                                        jnp.int32)
                    buk = lax.shift_right_logical(bits, 16)
                    hist = h0 if r % 2 == 0 else h1
                    plsc.addupdate_scatter(hist, [buk], ones,
                                           mask=bits != zerosv)

        _scan_chunks(acts_hbm, wid * _ROWS_PER_W, f, buf0, buf1, sem0, sem1,
                     process)

        @plsc.parallel_loop(0, _NBUK_A // _LANES, unroll=8)
        def _(i):
            sl = pl.ds(i * _LANES, _LANES)
            h0[sl] = h0[sl] + h1[sl]

        pltpu.sync_copy(h0, out_hbm.at[wid])

    return hist_a


def _make_hist_b(b, f):
    info = plsc.get_sparse_core_info()
    nw = info.num_cores * info.num_subcores

    @functools.partial(
        pl.kernel,
        mesh=_sc_mesh(),
        out_type=jax.ShapeDtypeStruct((nw, _NBUK), jnp.int32),
        compiler_params=pltpu.CompilerParams(needs_layout_passes=False),
        scratch_types=[
            pltpu.VMEM((_ROWS_PER_W, _CCOLS), jnp.float32),
            pltpu.VMEM((_ROWS_PER_W, _CCOLS), jnp.float32),
            pltpu.VMEM((_NBUK,), jnp.int32),
            pltpu.VMEM((_LANES,), jnp.int32),
            pltpu.SemaphoreType.DMA,
            pltpu.SemaphoreType.DMA,
        ],
    )
    def hist_b(acts_hbm, hstar_hbm, out_hbm, buf0, buf1, h0, hsv, sem0, sem1):
        wid = lax.axis_index("s") * info.num_cores + lax.axis_index("c")
        pltpu.sync_copy(hstar_hbm, hsv)
        h16 = hsv[...]
        ones = jnp.ones((_LANES,), jnp.int32)
        lowmask = jnp.full((_LANES,), 0xFFFF, jnp.int32)
        _zero_ref(h0, _NBUK)

        def process(buf):
            @plsc.parallel_loop(0, _CCOLS // _LANES, unroll=8)
            def _(j):
                for r in range(_ROWS_PER_W):
                    bits = plsc.bitcast(buf[r, pl.ds(j * _LANES, _LANES)],
                                        jnp.int32)
                    match = lax.shift_right_logical(bits, 16) == h16
                    buk = bits & lowmask
                    plsc.addupdate_scatter(h0, [buk], ones, mask=match)

        _scan_chunks(acts_hbm, wid * _ROWS_PER_W, f, buf0, buf1, sem0, sem1,
                     process)
        pltpu.sync_copy(h0, out_hbm.at[wid])

    return hist_b


def _find_rank(cnt, kb):
    n = cnt.shape[0]
    nb = 256
    blk = n // nb
    c2 = cnt.reshape(nb, blk)
    rowsum = c2.sum(axis=1)
    srow = jnp.cumsum(rowsum[::-1])[::-1]
    srow_pad = jnp.concatenate([srow, jnp.zeros((1,), srow.dtype)])
    i = jnp.sum(srow >= kb).astype(jnp.int32) - 1
    i0 = jnp.maximum(i, 0)
    crow = c2[i0]
    w = jnp.cumsum(crow[::-1])[::-1]
    w_pad = jnp.concatenate([w, jnp.zeros((1,), w.dtype)])
    tail = srow_pad[i0 + 1]
    j = jnp.sum(tail + w >= kb).astype(jnp.int32) - 1
    j0 = jnp.maximum(j, 0)
    idx = i0 * blk + j0
    r = kb - (tail + w_pad[j0 + 1])
    return jnp.where(i < 0, jnp.int32(-1), idx), r


def _rank_threshold(acts_parts, kb):
    b = acts_parts[0].shape[0]
    cnt_a = sum(_make_hist_a(b, a.shape[1])(a).sum(axis=0)
                for a in acts_parts)
    hstar, r = _find_rank(cnt_a, kb)

    h16 = jnp.full((_LANES,), jnp.maximum(hstar, 0), jnp.int32)
    cnt_b = sum(_make_hist_b(b, a.shape[1])(a, h16).sum(axis=0)
                for a in acts_parts)
    lstar, _ = _find_rank(cnt_b, r)

    vk_bits = jnp.left_shift(jnp.maximum(hstar, 0), 16) | jnp.maximum(lstar, 0)
    vk = lax.bitcast_convert_type(vk_bits, jnp.float32)
    return jnp.where(hstar < 0, jnp.float32(0.0), vk)


def _decode_body(vk_ref, acts_ref, w_ref, init_ref, out_ref):
    i = pl.program_id(0)
    vk = vk_ref[0, 0]
    a = acts_ref[...]
    m = jnp.where(a >= vk, a, 0.0)
    part = lax.dot_general(m, w_ref[...], (((1,), (1,)), ((), ())),
                           preferred_element_type=jnp.float32)

    @pl.when(i == 0)
    def _():
        out_ref[...] = jnp.broadcast_to(init_ref[...], out_ref.shape)

    out_ref[...] += part


def _decode(vk, acts, w_dec, init, ft, f_lo):
    b, fs = acts.shape
    d = w_dec.shape[0]
    off = f_lo // ft
    grid = (fs // ft,)
    return pl.pallas_call(
        _decode_body,
        grid=grid,
        in_specs=[
            pl.BlockSpec(memory_space=pltpu.SMEM),
            pl.BlockSpec((b, ft), lambda i: (0, i)),
            pl.BlockSpec((d, ft), lambda i: (0, i + off)),
            pl.BlockSpec(init.shape, lambda i: (0, 0)),
        ],
        out_specs=pl.BlockSpec((b, d), lambda i: (0, 0)),
        out_shape=jax.ShapeDtypeStruct((b, d), jnp.float32),
    )(vk.reshape(1, 1), acts, w_dec, init)


def kernel(x, W_enc, b_enc, W_dec, b_dec):
    b, d = x.shape
    f = W_enc.shape[0]
    kb = min(64 * b, b * f)
    ft = 2048
    acts = _encode(x, W_enc, b_enc, b_dec, ft, 0, f)
    vk = _rank_threshold([acts], kb)
    return _decode(vk, acts, W_dec, b_dec.reshape(1, d), ft, 0)
